# baseline (speedup 1.0000x reference)
import functools
import jax
import jax.numpy as jnp
from jax import lax
from jax.experimental import pallas as pl
from jax.experimental.pallas import tpu as pltpu
from jax.experimental.pallas import tpu_sc as plsc

f32 = jnp.float32
NN = 16384
NE = 32768
WN = 8192
WE = 32768
T = 16
R = 512

_INTERPRET = False


def _tc(body, out_shape, grid, in_specs, out_specs, scratch_shapes=()):
    return pl.pallas_call(
        body, out_shape=out_shape, grid=grid,
        in_specs=in_specs, out_specs=out_specs,
        scratch_shapes=list(scratch_shapes),
        interpret=_INTERPRET)


def _rowmap(i):
    return (i, 0)


def _partmap(i):
    return (0, i, 0)


def _constmap(*_):
    return (0, 0)


def _word_layer1(aggp, wh0, wdegp, Wg1p):
    def body(a_ref, w_ref, d_ref, wg_ref, o0_ref, o1_ref):
        deg = d_ref[0, :, 0:1] + d_ref[1, :, 0:1] + 1.0
        x = (a_ref[0] + a_ref[1] + w_ref[...]) / deg
        y = jnp.maximum(jnp.dot(x, wg_ref[...], preferred_element_type=f32), 0.0)
        o0_ref[...] = y[:, :128]
        o1_ref[...] = y[:, 128:]
    nb = WN // R
    return _tc(
        body,
        (jax.ShapeDtypeStruct((WN, 128), f32), jax.ShapeDtypeStruct((WN, 128), f32)),
        (nb,),
        [pl.BlockSpec((2, R, 128), _partmap),
         pl.BlockSpec((R, 128), _rowmap),
         pl.BlockSpec((2, R, 128), _partmap),
         pl.BlockSpec((128, 256), _constmap)],
        (pl.BlockSpec((R, 128), _rowmap), pl.BlockSpec((R, 128), _rowmap)),
    )(aggp, wh0, wdegp, Wg1p)


def _word_layer2(a0, a1, c0, c1, wdegp, Wg2, wgidf):
    def body(a0_ref, a1_ref, c0_ref, c1_ref, d_ref, wg_ref, g_ref,
             o0_ref, o1_ref, gw_ref, acc_ref):
        i = pl.program_id(0)
        deg = d_ref[0, :, 0:1] + d_ref[1, :, 0:1] + 1.0
        x0 = (a0_ref[0] + a0_ref[1] + c0_ref[...]) / deg
        x1 = (a1_ref[0] + a1_ref[1] + c1_ref[...]) / deg
        y = jnp.maximum(
            jnp.dot(x0, wg_ref[0:128, :], preferred_element_type=f32)
            + jnp.dot(x1, wg_ref[128:256, :], preferred_element_type=f32), 0.0)
        o0_ref[...] = y[:, :128]
        o1_ref[...] = y[:, 128:]

        @pl.when(i == 0)
        def _():
            acc_ref[...] = jnp.full((T, 256), -1e38, f32)
        g = g_ref[...]
        for t in range(T):
            m = g == float(t)
            acc_ref[t:t + 1, :] = jnp.maximum(
                acc_ref[t:t + 1, :],
                jnp.max(jnp.where(m, y, -1e38), axis=0, keepdims=True))

        @pl.when(i == pl.num_programs(0) - 1)
        def _():
            gw_ref[...] = jnp.where(acc_ref[...] < -1.0, 0.0, acc_ref[...])
    nb = WN // R
    return _tc(
        body,
        (jax.ShapeDtypeStruct((WN, 128), f32), jax.ShapeDtypeStruct((WN, 128), f32),
         jax.ShapeDtypeStruct((T, 256), f32)),
        (nb,),
        [pl.BlockSpec((2, R, 128), _partmap),
         pl.BlockSpec((2, R, 128), _partmap),
         pl.BlockSpec((R, 128), _rowmap),
         pl.BlockSpec((R, 128), _rowmap),
         pl.BlockSpec((2, R, 128), _partmap),
         pl.BlockSpec((256, 256), _constmap),
         pl.BlockSpec((R, 1), _rowmap)],
        (pl.BlockSpec((R, 128), _rowmap), pl.BlockSpec((R, 128), _rowmap),
         pl.BlockSpec((T, 256), _constmap)),
        [pltpu.VMEM((T, 256), f32)],
    )(a0, a1, c0, c1, wdegp, Wg2, wgidf)


def _rel_tables(rel_embeds, Wr1, Wr2):
    def body(rel_ref, w1_ref, w2_ref, nr0, nr1, ne_ref, e2_ref):
        rel = rel_ref[...]
        E1 = jnp.maximum(jnp.dot(rel, w1_ref[...], preferred_element_type=f32), 0.0)
        E2 = jnp.maximum(jnp.dot(E1, w2_ref[...], preferred_element_type=f32), 0.0)
        nr0[...] = -rel[:, 0:128]
        nr1[...] = -rel[:, 128:256]
        ne_ref[...] = -E1
        e2_ref[...] = E2
    shp = jax.ShapeDtypeStruct((256, 128), f32)
    return _tc(
        body,
        (shp, shp, shp, jax.ShapeDtypeStruct((256, 256), f32)),
        (1,),
        [pl.BlockSpec((256, 256), _constmap),
         pl.BlockSpec((256, 128), _constmap),
         pl.BlockSpec((128, 256), _constmap)],
        (pl.BlockSpec((256, 128), _constmap),) * 3
        + (pl.BlockSpec((256, 256), _constmap),),
    )(rel_embeds, Wr1, Wr2)


def _node_layer1(s1c0, s1c1, ndeg, hc0, hc1, Wc1, Wl1):
    def body(p0, p1, d_ref, h0, h1, wc_ref, wl_ref, o_ref):
        deg = d_ref[:, 0:1] + 1.0
        S = jnp.concatenate([p0[...], p1[...]], axis=1)
        hh = jnp.concatenate([h0[...], h1[...]], axis=1)
        agg = jnp.dot(S, wc_ref[...], preferred_element_type=f32) / deg
        o_ref[...] = jnp.maximum(
            agg + jnp.dot(hh, wl_ref[...], preferred_element_type=f32), 0.0)
    nb = NN // R
    return _tc(
        body,
        jax.ShapeDtypeStruct((NN, 128), f32),
        (nb,),
        [pl.BlockSpec((R, 128), _rowmap)] * 2
        + [pl.BlockSpec((R, 128), _rowmap)]
        + [pl.BlockSpec((R, 128), _rowmap)] * 2
        + [pl.BlockSpec((256, 128), _constmap), pl.BlockSpec((256, 128), _constmap)],
        pl.BlockSpec((R, 128), _rowmap),
    )(s1c0, s1c1, ndeg, hc0, hc1, Wc1, Wl1)


def _node_layer2(s2, ndeg, h1, Wc2, Wl2):
    def body(q_ref, d_ref, h_ref, wc_ref, wl_ref, o_ref):
        deg = d_ref[:, 0:1] + 1.0
        agg = jnp.dot(q_ref[...], wc_ref[...], preferred_element_type=f32) / deg
        o_ref[...] = jnp.maximum(
            agg + jnp.dot(h_ref[...], wl_ref[...], preferred_element_type=f32), 0.0)
    nb = NN // R
    return _tc(
        body,
        jax.ShapeDtypeStruct((NN, 256), f32),
        (nb,),
        [pl.BlockSpec((R, 128), _rowmap)] * 3
        + [pl.BlockSpec((128, 256), _constmap), pl.BlockSpec((128, 256), _constmap)],
        pl.BlockSpec((R, 256), _rowmap),
    )(s2, ndeg, h1, Wc2, Wl2)


def _attn_gmax(Q, Hlo, Hhi, gidf, Wa, Wo, hm_off):
    NQ = Q.shape[0]
    nb = NQ // R

    def hmmap(i):
        return (i + hm_off, 0)

    def body(q_ref, hlo_ref, hhi_ref, g_ref, wa_ref, wo_ref, out_ref, acc_ref):
        i = pl.program_id(0)
        Qb = q_ref[...]
        QW = jnp.dot(Qb, wa_ref[...], preferred_element_type=f32)
        Hlo_b = hlo_ref[...].reshape(R, 4, 128)
        Hhi_b = hhi_ref[...].reshape(R, 4, 128)
        QWlo = QW[:, :128]
        QWhi = QW[:, 128:]
        sc = [jnp.sum(QWlo * Hlo_b[:, m, :], axis=1, keepdims=True)
              + jnp.sum(QWhi * Hhi_b[:, m, :], axis=1, keepdims=True)
              for m in range(4)]
        smax = jnp.maximum(jnp.maximum(sc[0], sc[1]), jnp.maximum(sc[2], sc[3]))
        ew = [jnp.exp(s - smax) for s in sc]
        tot = ew[0] + ew[1] + ew[2] + ew[3]
        ctx_lo = sum((ew[m] / tot) * Hlo_b[:, m, :] for m in range(4))
        ctx_hi = sum((ew[m] / tot) * Hhi_b[:, m, :] for m in range(4))
        y = jnp.tanh(
            jnp.dot(ctx_lo, wo_ref[0:128, :], preferred_element_type=f32)
            + jnp.dot(ctx_hi, wo_ref[128:256, :], preferred_element_type=f32)
            + jnp.dot(Qb, wo_ref[256:512, :], preferred_element_type=f32))

        @pl.when(i == 0)
        def _():
            acc_ref[...] = jnp.full((T, 256), -1e38, f32)
        g = g_ref[...]
        for t in range(T):
            m = g == float(t)
            acc_ref[t:t + 1, :] = jnp.maximum(
                acc_ref[t:t + 1, :],
                jnp.max(jnp.where(m, y, -1e38), axis=0, keepdims=True))

        @pl.when(i == pl.num_programs(0) - 1)
        def _():
            out_ref[...] = jnp.where(acc_ref[...] < -2.0, 0.0, acc_ref[...])

    return _tc(
        body,
        jax.ShapeDtypeStruct((T, 256), f32),
        (nb,),
        [pl.BlockSpec((R, 256), _rowmap),
         pl.BlockSpec((4 * R, 128), hmmap),
         pl.BlockSpec((4 * R, 128), hmmap),
         pl.BlockSpec((R, 1), _rowmap),
         pl.BlockSpec((256, 256), _constmap),
         pl.BlockSpec((512, 256), _constmap)],
        pl.BlockSpec((T, 256), _constmap),
        [pltpu.VMEM((T, 256), f32)],
    )(Q, Hlo, Hhi, gidf, Wa, Wo)


def _final(gn, ge, gw, onehot):
    def body(oh_ref, gn_ref, ge_ref, gw_ref, o_ref):
        oh = oh_ref[...]
        o_ref[:, 0:256] = jnp.dot(oh, gn_ref[...], preferred_element_type=f32)
        o_ref[:, 256:512] = jnp.dot(oh, ge_ref[...], preferred_element_type=f32)
        o_ref[:, 512:768] = jnp.dot(oh, gw_ref[...], preferred_element_type=f32)
    return _tc(
        body,
        jax.ShapeDtypeStruct((640, 768), f32),
        (1,),
        [pl.BlockSpec((640, 16), _constmap)]
        + [pl.BlockSpec((T, 256), _constmap)] * 3,
        pl.BlockSpec((640, 768), _constmap),
    )(onehot, gn, ge, gw)


NW = 32
CH = 128
ZR = 128


def _sc_mesh():
    return plsc.VectorSubcoreMesh(core_axis_name="c", subcore_axis_name="s")


def _zero_fill(zbuf, C):
    def zrow(r, _):
        for kk in range(C // 16):
            zbuf[r, pl.ds(kk * 16, 16)] = jnp.zeros((16,), f32)
        return 0
    lax.fori_loop(0, ZR, zrow, 0)


def _sc_gather_multi(tables, idx):
    B = idx.shape[0]
    b_per_w = B // NW
    n_ch = b_per_w // CH
    nt = len(tables)
    assert n_ch % 2 == 0
    idx2d = idx.reshape(B // CH, CH)
    scratch = [pltpu.VMEM((n_ch, CH), jnp.int32)]
    scratch += [pltpu.VMEM((2, CH, t.shape[1]), f32) for t in tables]
    scratch += [pltpu.SemaphoreType.DMA] * 4

    @functools.partial(
        pl.kernel, mesh=_sc_mesh(),
        out_type=tuple(jax.ShapeDtypeStruct((B, t.shape[1]), f32) for t in tables),
        scratch_types=scratch)
    def k(*refs):
        tbls = refs[:nt]
        idx_hbm = refs[nt]
        outs = refs[nt + 1:2 * nt + 1]
        idx_v = refs[2 * nt + 1]
        rows = refs[2 * nt + 2:3 * nt + 2]
        gsem = refs[3 * nt + 2:3 * nt + 4]
        wsem = refs[3 * nt + 4:3 * nt + 6]
        wid = lax.axis_index("s") * 2 + lax.axis_index("c")
        pltpu.sync_copy(idx_hbm.at[pl.ds(wid * n_ch, n_ch)], idx_v)

        def fire_g(i, b):
            for j in range(nt):
                pltpu.async_copy(tbls[j].at[idx_v.at[i]], rows[j].at[b], gsem[b])

        def drain_g(b):
            for j in range(nt):
                pltpu.make_async_copy(tbls[j].at[idx_v.at[0]],
                                      rows[j].at[b], gsem[b]).wait()

        def fire_w(i, b):
            off = (wid * n_ch + i) * CH
            for j in range(nt):
                pltpu.async_copy(rows[j].at[b], outs[j].at[pl.ds(off, CH)], wsem[b])

        def drain_w(b):
            for j in range(nt):
                pltpu.make_async_copy(rows[j].at[b],
                                      outs[j].at[pl.ds(0, CH)], wsem[b]).wait()

        fire_g(0, 0)

        def body(hh, _):
            g0 = 2 * hh
            fire_g(g0 + 1, 1)
            drain_g(0)
            fire_w(g0, 0)
            drain_g(1)
            fire_w(g0 + 1, 1)
            drain_w(0)

            @pl.when(hh + 1 < n_ch // 2)
            def _():
                fire_g(g0 + 2, 0)
            drain_w(1)
            return 0
        lax.fori_loop(0, n_ch // 2, body, 0)

    res = k(*tables, idx2d)
    return list(res) if isinstance(res, (tuple, list)) else [res]


def _sc_gather(table, idx):
    return _sc_gather_multi([table], idx)[0]


def _sc_segsum_part(tA, sA, dst, N):
    E = dst.shape[0]
    C = tA.shape[1]
    e_per_w = E // NW
    n_ch = e_per_w // CH
    rows_per = N // 16
    n_z = rows_per // ZR

    scratch = [pltpu.VMEM((n_ch, CH), jnp.int32),
               pltpu.VMEM((n_ch, CH), jnp.int32),
               pltpu.VMEM((CH, C), f32),
               pltpu.VMEM((CH, C), f32),
               pltpu.VMEM((ZR, C), f32),
               pltpu.VMEM_SHARED((N, C), f32),
               pltpu.SemaphoreType.DMA]

    @functools.partial(
        pl.kernel, mesh=_sc_mesh(),
        out_type=jax.ShapeDtypeStruct((2, N, C), f32),
        scratch_types=scratch)
    def k(tA_h, sA_h, dst_h, out_h, iva, ivd, ra0, ra1, zbuf, acc, sem):
        c = lax.axis_index("c")
        s = lax.axis_index("s")
        _zero_fill(zbuf, C)

        def zcopy(j, _):
            pltpu.sync_copy(zbuf, acc.at[pl.ds(s * rows_per + j * ZR, ZR)])
            return 0
        lax.fori_loop(0, n_z, zcopy, 0)
        plsc.subcore_barrier()

        crow = (c * 16 + s) * n_ch
        pltpu.sync_copy(sA_h.at[pl.ds(crow, n_ch)], iva)
        pltpu.sync_copy(dst_h.at[pl.ds(crow, n_ch)], ivd)

        def body(g, _):
            i0 = 2 * g
            d0 = pltpu.async_copy(tA_h.at[iva.at[i0]], ra0, sem)
            d1 = pltpu.async_copy(tA_h.at[iva.at[i0 + 1]], ra1, sem)
            d0.wait()
            pltpu.sync_copy(ra0, acc.at[ivd.at[i0]], add=True)
            d1.wait()
            pltpu.sync_copy(ra1, acc.at[ivd.at[i0 + 1]], add=True)
            return 0
        lax.fori_loop(0, n_ch // 2, body, 0)
        plsc.subcore_barrier()
        pltpu.sync_copy(acc.at[pl.ds(s * rows_per, rows_per)],
                        out_h.at[c, pl.ds(s * rows_per, rows_per)])

    return k(tA, sA.reshape(-1, CH), dst.reshape(-1, CH))


def _sc_segsum_range(tA, sA, tB, sB, dst, N):
    E = dst.shape[0]
    C = tA.shape[1]
    half = N // 2
    e_per_w = E // 16
    CHR = 64
    n_ch = e_per_w // CHR
    rows_per = half // 16
    n_z = rows_per // ZR

    scratch = [pltpu.VMEM((n_ch, CHR), jnp.int32),
               pltpu.VMEM((n_ch, CHR), jnp.int32),
               pltpu.VMEM((n_ch, CHR), jnp.int32),
               pltpu.VMEM((CHR, C), f32),
               pltpu.VMEM((CHR, C), f32),
               pltpu.VMEM((CHR, C), f32),
               pltpu.VMEM((CHR, C), f32),
               pltpu.VMEM((ZR, C), f32),
               pltpu.VMEM_SHARED((half + 8, C), f32),
               pltpu.SemaphoreType.DMA]

    @functools.partial(
        pl.kernel, mesh=_sc_mesh(),
        out_type=jax.ShapeDtypeStruct((N, C), f32),
        scratch_types=scratch)
    def k(tA_h, sA_h, tB_h, sB_h, dst_h, out_h,
          iva, ivb, ivd, ra0, ra1, rb0, rb1, zbuf, acc, sem):
        c = lax.axis_index("c")
        s = lax.axis_index("s")
        _zero_fill(zbuf, C)

        def zcopy(j, _):
            pltpu.sync_copy(zbuf, acc.at[pl.ds(s * rows_per + j * ZR, ZR)])
            return 0
        lax.fori_loop(0, n_z, zcopy, 0)
        plsc.subcore_barrier()

        lo = c * half
        crow = s * n_ch
        pltpu.sync_copy(sA_h.at[pl.ds(crow, n_ch)], iva)
        pltpu.sync_copy(sB_h.at[pl.ds(crow, n_ch)], ivb)
        pltpu.sync_copy(dst_h.at[pl.ds(crow, n_ch)], ivd)

        def remap(r, _):
            for kk in range(CHR // 16):
                sl = pl.ds(kk * 16, 16)
                v = ivd[r, sl] - lo
                inr = (v >= 0) & (v < half)
                ivd[r, sl] = jnp.where(inr, v, half)
            return 0
        lax.fori_loop(0, n_ch, remap, 0)

        def body(g, _):
            i0 = 2 * g
            d0a = pltpu.async_copy(tA_h.at[iva.at[i0]], ra0, sem)
            d0b = pltpu.async_copy(tB_h.at[ivb.at[i0]], rb0, sem)
            d1a = pltpu.async_copy(tA_h.at[iva.at[i0 + 1]], ra1, sem)
            d1b = pltpu.async_copy(tB_h.at[ivb.at[i0 + 1]], rb1, sem)
            d0a.wait()
            d0b.wait()
            pltpu.sync_copy(ra0, acc.at[ivd.at[i0]], add=True)
            pltpu.sync_copy(rb0, acc.at[ivd.at[i0]], add=True)
            d1a.wait()
            d1b.wait()
            pltpu.sync_copy(ra1, acc.at[ivd.at[i0 + 1]], add=True)
            pltpu.sync_copy(rb1, acc.at[ivd.at[i0 + 1]], add=True)
            return 0
        lax.fori_loop(0, n_ch // 2, body, 0)
        plsc.subcore_barrier()
        pltpu.sync_copy(acc.at[pl.ds(s * rows_per, rows_per)],
                        out_h.at[pl.ds(c * half + s * rows_per, rows_per)])

    return k(tA, sA.reshape(-1, CHR), tB, sB.reshape(-1, CHR), dst.reshape(-1, CHR))


def _sc_degree_part(dst, N):
    E = dst.shape[0]
    C = 128
    e_per_w = E // NW
    n_ch = e_per_w // CH
    rows_per = N // 16
    n_z = rows_per // ZR

    scratch = [pltpu.VMEM((CH,), jnp.int32),
               pltpu.VMEM((CH, C), f32),
               pltpu.VMEM((ZR, C), f32),
               pltpu.VMEM_SHARED((N, C), f32),
               pltpu.SemaphoreType.DMA]

    @functools.partial(
        pl.kernel, mesh=_sc_mesh(),
        out_type=jax.ShapeDtypeStruct((2, N, C), f32),
        scratch_types=scratch)
    def k(dst_h, out_h, ivd, obuf, zbuf, acc, sem):
        c = lax.axis_index("c")
        s = lax.axis_index("s")
        _zero_fill(zbuf, C)

        def fill(r, _):
            for kk in range(C // 16):
                obuf[r, pl.ds(kk * 16, 16)] = jnp.ones((16,), f32)
            return 0
        lax.fori_loop(0, ZR, fill, 0)

        def zcopy(j, _):
            pltpu.sync_copy(zbuf, acc.at[pl.ds(s * rows_per + j * ZR, ZR)])
            return 0
        lax.fori_loop(0, n_z, zcopy, 0)
        plsc.subcore_barrier()

        base = (c * 16 + s) * e_per_w

        def body(i, _):
            off = base + i * CH
            pltpu.sync_copy(dst_h.at[pl.ds(off, CH)], ivd)
            pltpu.sync_copy(obuf, acc.at[ivd], add=True)
            return 0
        lax.fori_loop(0, n_ch, body, 0)
        plsc.subcore_barrier()
        pltpu.sync_copy(acc.at[pl.ds(s * rows_per, rows_per)],
                        out_h.at[c, pl.ds(s * rows_per, rows_per)])

    return k(dst)


def _sc_degree_range(dst, N):
    E = dst.shape[0]
    C = 128
    half = N // 2
    e_per_w = E // 16
    n_ch = e_per_w // CH
    rows_per = half // 16
    n_z = rows_per // ZR

    scratch = [pltpu.VMEM((CH,), jnp.int32),
               pltpu.VMEM((CH, C), f32),
               pltpu.VMEM((ZR, C), f32),
               pltpu.VMEM_SHARED((half + 8, C), f32),
               pltpu.SemaphoreType.DMA]

    @functools.partial(
        pl.kernel, mesh=_sc_mesh(),
        out_type=jax.ShapeDtypeStruct((N, C), f32),
        scratch_types=scratch)
    def k(dst_h, out_h, ivd, obuf, zbuf, acc, sem):
        c = lax.axis_index("c")
        s = lax.axis_index("s")
        _zero_fill(zbuf, C)

        def fill(r, _):
            for kk in range(C // 16):
                obuf[r, pl.ds(kk * 16, 16)] = jnp.ones((16,), f32)
            return 0
        lax.fori_loop(0, ZR, fill, 0)

        def zcopy(j, _):
            pltpu.sync_copy(zbuf, acc.at[pl.ds(s * rows_per + j * ZR, ZR)])
            return 0
        lax.fori_loop(0, n_z, zcopy, 0)
        plsc.subcore_barrier()

        lo = c * half
        base = s * e_per_w

        def body(i, _):
            off = base + i * CH
            pltpu.sync_copy(dst_h.at[pl.ds(off, CH)], ivd)
            for kk in range(CH // 16):
                sl = pl.ds(kk * 16, 16)
                v = ivd[sl] - lo
                inr = (v >= 0) & (v < half)
                ivd[sl] = jnp.where(inr, v, half)
            pltpu.sync_copy(obuf, acc.at[ivd], add=True)
            return 0
        lax.fori_loop(0, n_ch, body, 0)
        plsc.subcore_barrier()
        pltpu.sync_copy(acc.at[pl.ds(s * rows_per, rows_per)],
                        out_h.at[pl.ds(c * half + s * rows_per, rows_per)])

    return k(dst)


def kernel(ent_embeds, rel_embeds, word_embeds, W_gcn1, W_gcn2, W_comp1, W_loop1,
           W_rel1, W_comp2, W_loop2, W_rel2, W_attn, W_out,
           node_ids, edge_src, edge_dst, edge_type, node_graph_ids, edge_graph_ids,
           wg_node_ids, wg_src, wg_dst, wg_graph_ids, word_query_idx, time_idx):
    wp = jnp.pad(word_embeds, ((0, 0), (0, 28)))
    Wg1p = jnp.pad(W_gcn1, ((0, 28), (0, 0)))
    ent_c = [ent_embeds[:, 128 * k:128 * (k + 1)] for k in range(2)]
    node_gidf = node_graph_ids.astype(f32)[:, None]
    edge_gidf = edge_graph_ids.astype(f32)[:, None]
    wg_gidf = wg_graph_ids.astype(f32)[:, None]
    onehot = (time_idx.reshape(-1, 1) == jnp.arange(T)[None, :]).astype(f32)
    wqi = word_query_idx.reshape(-1)

    hc = _sc_gather_multi(ent_c, node_ids)
    wh0 = _sc_gather(wp, wg_node_ids)
    wdegp = _sc_degree_part(wg_dst, WN)
    ndeg = _sc_degree_range(edge_dst, NN)

    wa0 = _sc_segsum_part(wh0, wg_src, wg_dst, WN)
    wh1c0, wh1c1 = _word_layer1(wa0, wh0, wdegp, Wg1p)
    wa1_0 = _sc_segsum_part(wh1c0, wg_src, wg_dst, WN)
    wa1_1 = _sc_segsum_part(wh1c1, wg_src, wg_dst, WN)
    wh2c0, wh2c1, gw = _word_layer2(wa1_0, wa1_1, wh1c0, wh1c1, wdegp, W_gcn2, wg_gidf)

    nr0, nr1, ne1, E2t = _rel_tables(rel_embeds, W_rel1, W_rel2)

    s1c0 = _sc_segsum_range(hc[0], edge_src, nr0, edge_type, edge_dst, NN)
    s1c1 = _sc_segsum_range(hc[1], edge_src, nr1, edge_type, edge_dst, NN)
    h1 = _node_layer1(s1c0, s1c1, ndeg, hc[0], hc[1], W_comp1, W_loop1)
    s2 = _sc_segsum_range(h1, edge_src, ne1, edge_type, edge_dst, NN)
    h2 = _node_layer2(s2, ndeg, h1, W_comp2, W_loop2)
    e2 = _sc_gather(E2t, edge_type)

    Hlo, Hhi = _sc_gather_multi([wh2c0, wh2c1], wqi)
    gn = _attn_gmax(h2, Hlo, Hhi, node_gidf, W_attn, W_out, 0)
    ge = _attn_gmax(e2, Hlo, Hhi, edge_gidf, W_attn, W_out, NN // R)

    out640 = _final(gn, ge, gw, onehot)
    return out640.reshape(time_idx.shape[0], time_idx.shape[1], 3 * 256)


# speedup vs baseline: 1.0758x; 1.0758x over previous
import functools
import jax
import jax.numpy as jnp
from jax import lax
from jax.experimental import pallas as pl
from jax.experimental.pallas import tpu as pltpu
from jax.experimental.pallas import tpu_sc as plsc

f32 = jnp.float32
NN = 16384
NE = 32768
WN = 8192
WE = 32768
T = 16
R = 512

_INTERPRET = False


def _tc(body, out_shape, grid, in_specs, out_specs, scratch_shapes=()):
    return pl.pallas_call(
        body, out_shape=out_shape, grid=grid,
        in_specs=in_specs, out_specs=out_specs,
        scratch_shapes=list(scratch_shapes),
        interpret=_INTERPRET)


def _rowmap(i):
    return (i, 0)


def _partmap(i):
    return (0, i, 0)


def _constmap(*_):
    return (0, 0)


def _word_layer1(aggp, wh0, wdegp, Wg1p):
    def body(a_ref, w_ref, d_ref, wg_ref, o0_ref, o1_ref):
        deg = d_ref[0, :, 0:1] + d_ref[1, :, 0:1] + 1.0
        x = (a_ref[0] + a_ref[1] + w_ref[...]) / deg
        y = jnp.maximum(jnp.dot(x, wg_ref[...], preferred_element_type=f32), 0.0)
        o0_ref[...] = y[:, :128]
        o1_ref[...] = y[:, 128:]
    nb = WN // R
    return _tc(
        body,
        (jax.ShapeDtypeStruct((WN, 128), f32), jax.ShapeDtypeStruct((WN, 128), f32)),
        (nb,),
        [pl.BlockSpec((2, R, 128), _partmap),
         pl.BlockSpec((R, 128), _rowmap),
         pl.BlockSpec((2, R, 128), _partmap),
         pl.BlockSpec((128, 256), _constmap)],
        (pl.BlockSpec((R, 128), _rowmap), pl.BlockSpec((R, 128), _rowmap)),
    )(aggp, wh0, wdegp, Wg1p)


def _word_layer2(a0, a1, c0, c1, wdegp, Wg2, wgidf):
    def body(a0_ref, a1_ref, c0_ref, c1_ref, d_ref, wg_ref, g_ref,
             o0_ref, o1_ref, gw_ref, acc_ref):
        i = pl.program_id(0)
        deg = d_ref[0, :, 0:1] + d_ref[1, :, 0:1] + 1.0
        x0 = (a0_ref[0] + a0_ref[1] + c0_ref[...]) / deg
        x1 = (a1_ref[0] + a1_ref[1] + c1_ref[...]) / deg
        y = jnp.maximum(
            jnp.dot(x0, wg_ref[0:128, :], preferred_element_type=f32)
            + jnp.dot(x1, wg_ref[128:256, :], preferred_element_type=f32), 0.0)
        o0_ref[...] = y[:, :128]
        o1_ref[...] = y[:, 128:]

        @pl.when(i == 0)
        def _():
            acc_ref[...] = jnp.full((T, 256), -1e38, f32)
        g = g_ref[...]
        gmin = jnp.min(g)
        gmax = jnp.max(g)
        for t in range(T):
            @pl.when(jnp.logical_and(gmin <= float(t), float(t) <= gmax))
            def _():
                m = g == float(t)
                acc_ref[t:t + 1, :] = jnp.maximum(
                    acc_ref[t:t + 1, :],
                    jnp.max(jnp.where(m, y, -1e38), axis=0, keepdims=True))

        @pl.when(i == pl.num_programs(0) - 1)
        def _():
            gw_ref[...] = jnp.where(acc_ref[...] < -1.0, 0.0, acc_ref[...])
    nb = WN // R
    return _tc(
        body,
        (jax.ShapeDtypeStruct((WN, 128), f32), jax.ShapeDtypeStruct((WN, 128), f32),
         jax.ShapeDtypeStruct((T, 256), f32)),
        (nb,),
        [pl.BlockSpec((2, R, 128), _partmap),
         pl.BlockSpec((2, R, 128), _partmap),
         pl.BlockSpec((R, 128), _rowmap),
         pl.BlockSpec((R, 128), _rowmap),
         pl.BlockSpec((2, R, 128), _partmap),
         pl.BlockSpec((256, 256), _constmap),
         pl.BlockSpec((R, 1), _rowmap)],
        (pl.BlockSpec((R, 128), _rowmap), pl.BlockSpec((R, 128), _rowmap),
         pl.BlockSpec((T, 256), _constmap)),
        [pltpu.VMEM((T, 256), f32)],
    )(a0, a1, c0, c1, wdegp, Wg2, wgidf)


def _rel_tables(rel_embeds, Wr1, Wr2):
    def body(rel_ref, w1_ref, w2_ref, nr0, nr1, ne_ref, e2_ref):
        rel = rel_ref[...]
        E1 = jnp.maximum(jnp.dot(rel, w1_ref[...], preferred_element_type=f32), 0.0)
        E2 = jnp.maximum(jnp.dot(E1, w2_ref[...], preferred_element_type=f32), 0.0)
        nr0[...] = -rel[:, 0:128]
        nr1[...] = -rel[:, 128:256]
        ne_ref[...] = -E1
        e2_ref[...] = E2
    shp = jax.ShapeDtypeStruct((256, 128), f32)
    return _tc(
        body,
        (shp, shp, shp, jax.ShapeDtypeStruct((256, 256), f32)),
        (1,),
        [pl.BlockSpec((256, 256), _constmap),
         pl.BlockSpec((256, 128), _constmap),
         pl.BlockSpec((128, 256), _constmap)],
        (pl.BlockSpec((256, 128), _constmap),) * 3
        + (pl.BlockSpec((256, 256), _constmap),),
    )(rel_embeds, Wr1, Wr2)


def _node_layer1(s1c0, s1c1, ndeg, hc0, hc1, Wc1, Wl1):
    def body(p0, p1, d_ref, h0, h1, wc_ref, wl_ref, o_ref):
        deg = d_ref[:, 0:1] + 1.0
        S = jnp.concatenate([p0[...], p1[...]], axis=1)
        hh = jnp.concatenate([h0[...], h1[...]], axis=1)
        agg = jnp.dot(S, wc_ref[...], preferred_element_type=f32) / deg
        o_ref[...] = jnp.maximum(
            agg + jnp.dot(hh, wl_ref[...], preferred_element_type=f32), 0.0)
    nb = NN // R
    return _tc(
        body,
        jax.ShapeDtypeStruct((NN, 128), f32),
        (nb,),
        [pl.BlockSpec((R, 128), _rowmap)] * 2
        + [pl.BlockSpec((R, 128), _rowmap)]
        + [pl.BlockSpec((R, 128), _rowmap)] * 2
        + [pl.BlockSpec((256, 128), _constmap), pl.BlockSpec((256, 128), _constmap)],
        pl.BlockSpec((R, 128), _rowmap),
    )(s1c0, s1c1, ndeg, hc0, hc1, Wc1, Wl1)


def _node_layer2(s2, ndeg, h1, Wc2, Wl2):
    def body(q_ref, d_ref, h_ref, wc_ref, wl_ref, o_ref):
        deg = d_ref[:, 0:1] + 1.0
        agg = jnp.dot(q_ref[...], wc_ref[...], preferred_element_type=f32) / deg
        o_ref[...] = jnp.maximum(
            agg + jnp.dot(h_ref[...], wl_ref[...], preferred_element_type=f32), 0.0)
    nb = NN // R
    return _tc(
        body,
        jax.ShapeDtypeStruct((NN, 256), f32),
        (nb,),
        [pl.BlockSpec((R, 128), _rowmap)] * 3
        + [pl.BlockSpec((128, 256), _constmap), pl.BlockSpec((128, 256), _constmap)],
        pl.BlockSpec((R, 256), _rowmap),
    )(s2, ndeg, h1, Wc2, Wl2)


def _attn_gmax(Q, Hlo, Hhi, gidf, Wa, Wo, hm_off):
    NQ = Q.shape[0]
    nb = NQ // R

    def hmmap(i):
        return (i + hm_off, 0)

    def body(q_ref, hlo_ref, hhi_ref, g_ref, wa_ref, wo_ref, out_ref, acc_ref):
        i = pl.program_id(0)
        Qb = q_ref[...]
        QW = jnp.dot(Qb, wa_ref[...], preferred_element_type=f32)
        Hlo_b = hlo_ref[...].reshape(R, 4, 128)
        Hhi_b = hhi_ref[...].reshape(R, 4, 128)
        QWlo = QW[:, :128]
        QWhi = QW[:, 128:]
        sc = [jnp.sum(QWlo * Hlo_b[:, m, :], axis=1, keepdims=True)
              + jnp.sum(QWhi * Hhi_b[:, m, :], axis=1, keepdims=True)
              for m in range(4)]
        smax = jnp.maximum(jnp.maximum(sc[0], sc[1]), jnp.maximum(sc[2], sc[3]))
        ew = [jnp.exp(s - smax) for s in sc]
        tot = ew[0] + ew[1] + ew[2] + ew[3]
        ctx_lo = sum((ew[m] / tot) * Hlo_b[:, m, :] for m in range(4))
        ctx_hi = sum((ew[m] / tot) * Hhi_b[:, m, :] for m in range(4))
        y = jnp.tanh(
            jnp.dot(ctx_lo, wo_ref[0:128, :], preferred_element_type=f32)
            + jnp.dot(ctx_hi, wo_ref[128:256, :], preferred_element_type=f32)
            + jnp.dot(Qb, wo_ref[256:512, :], preferred_element_type=f32))

        @pl.when(i == 0)
        def _():
            acc_ref[...] = jnp.full((T, 256), -1e38, f32)
        g = g_ref[...]
        gmin = jnp.min(g)
        gmax = jnp.max(g)
        for t in range(T):
            @pl.when(jnp.logical_and(gmin <= float(t), float(t) <= gmax))
            def _():
                m = g == float(t)
                acc_ref[t:t + 1, :] = jnp.maximum(
                    acc_ref[t:t + 1, :],
                    jnp.max(jnp.where(m, y, -1e38), axis=0, keepdims=True))

        @pl.when(i == pl.num_programs(0) - 1)
        def _():
            out_ref[...] = jnp.where(acc_ref[...] < -2.0, 0.0, acc_ref[...])

    return _tc(
        body,
        jax.ShapeDtypeStruct((T, 256), f32),
        (nb,),
        [pl.BlockSpec((R, 256), _rowmap),
         pl.BlockSpec((4 * R, 128), hmmap),
         pl.BlockSpec((4 * R, 128), hmmap),
         pl.BlockSpec((R, 1), _rowmap),
         pl.BlockSpec((256, 256), _constmap),
         pl.BlockSpec((512, 256), _constmap)],
        pl.BlockSpec((T, 256), _constmap),
        [pltpu.VMEM((T, 256), f32)],
    )(Q, Hlo, Hhi, gidf, Wa, Wo)


def _final(gn, ge, gw, onehot):
    def body(oh_ref, gn_ref, ge_ref, gw_ref, o_ref):
        oh = oh_ref[...]
        o_ref[:, 0:256] = jnp.dot(oh, gn_ref[...], preferred_element_type=f32)
        o_ref[:, 256:512] = jnp.dot(oh, ge_ref[...], preferred_element_type=f32)
        o_ref[:, 512:768] = jnp.dot(oh, gw_ref[...], preferred_element_type=f32)
    return _tc(
        body,
        jax.ShapeDtypeStruct((640, 768), f32),
        (1,),
        [pl.BlockSpec((640, 16), _constmap)]
        + [pl.BlockSpec((T, 256), _constmap)] * 3,
        pl.BlockSpec((640, 768), _constmap),
    )(onehot, gn, ge, gw)


NW = 32
CH = 128
ZR = 128


def _sc_mesh():
    return plsc.VectorSubcoreMesh(core_axis_name="c", subcore_axis_name="s")


def _zero_fill(zbuf, C):
    def zrow(r, _):
        for kk in range(C // 16):
            zbuf[r, pl.ds(kk * 16, 16)] = jnp.zeros((16,), f32)
        return 0
    lax.fori_loop(0, ZR, zrow, 0)


def _sc_gather_multi(tables, idx):
    B = idx.shape[0]
    b_per_w = B // NW
    n_ch = b_per_w // CH
    nt = len(tables)
    assert n_ch % 2 == 0
    idx2d = idx.reshape(B // CH, CH)
    scratch = [pltpu.VMEM((n_ch, CH), jnp.int32)]
    scratch += [pltpu.VMEM((2, CH, t.shape[1]), f32) for t in tables]
    scratch += [pltpu.SemaphoreType.DMA] * 4

    @functools.partial(
        pl.kernel, mesh=_sc_mesh(),
        out_type=tuple(jax.ShapeDtypeStruct((B, t.shape[1]), f32) for t in tables),
        scratch_types=scratch)
    def k(*refs):
        tbls = refs[:nt]
        idx_hbm = refs[nt]
        outs = refs[nt + 1:2 * nt + 1]
        idx_v = refs[2 * nt + 1]
        rows = refs[2 * nt + 2:3 * nt + 2]
        gsem = refs[3 * nt + 2:3 * nt + 4]
        wsem = refs[3 * nt + 4:3 * nt + 6]
        wid = lax.axis_index("s") * 2 + lax.axis_index("c")
        pltpu.sync_copy(idx_hbm.at[pl.ds(wid * n_ch, n_ch)], idx_v)

        def fire_g(i, b):
            for j in range(nt):
                pltpu.async_copy(tbls[j].at[idx_v.at[i]], rows[j].at[b], gsem[b])

        def drain_g(b):
            for j in range(nt):
                pltpu.make_async_copy(tbls[j].at[idx_v.at[0]],
                                      rows[j].at[b], gsem[b]).wait()

        def fire_w(i, b):
            off = (wid * n_ch + i) * CH
            for j in range(nt):
                pltpu.async_copy(rows[j].at[b], outs[j].at[pl.ds(off, CH)], wsem[b])

        def drain_w(b):
            for j in range(nt):
                pltpu.make_async_copy(rows[j].at[b],
                                      outs[j].at[pl.ds(0, CH)], wsem[b]).wait()

        fire_g(0, 0)

        def body(hh, _):
            g0 = 2 * hh
            fire_g(g0 + 1, 1)
            drain_g(0)
            fire_w(g0, 0)
            drain_g(1)
            fire_w(g0 + 1, 1)
            drain_w(0)

            @pl.when(hh + 1 < n_ch // 2)
            def _():
                fire_g(g0 + 2, 0)
            drain_w(1)
            return 0
        lax.fori_loop(0, n_ch // 2, body, 0)

    res = k(*tables, idx2d)
    return list(res) if isinstance(res, (tuple, list)) else [res]


def _sc_gather(table, idx):
    return _sc_gather_multi([table], idx)[0]


def _sc_segsum_part(tA, sA, dst, N):
    E = dst.shape[0]
    C = tA.shape[1]
    e_per_w = E // NW
    n_ch = e_per_w // CH
    rows_per = N // 16
    n_z = rows_per // ZR

    scratch = [pltpu.VMEM((n_ch, CH), jnp.int32),
               pltpu.VMEM((n_ch, CH), jnp.int32),
               pltpu.VMEM((CH, C), f32),
               pltpu.VMEM((CH, C), f32),
               pltpu.VMEM((ZR, C), f32),
               pltpu.VMEM_SHARED((N, C), f32),
               pltpu.SemaphoreType.DMA]

    @functools.partial(
        pl.kernel, mesh=_sc_mesh(),
        out_type=jax.ShapeDtypeStruct((2, N, C), f32),
        scratch_types=scratch)
    def k(tA_h, sA_h, dst_h, out_h, iva, ivd, ra0, ra1, zbuf, acc, sem):
        c = lax.axis_index("c")
        s = lax.axis_index("s")
        _zero_fill(zbuf, C)

        def zcopy(j, _):
            pltpu.sync_copy(zbuf, acc.at[pl.ds(s * rows_per + j * ZR, ZR)])
            return 0
        lax.fori_loop(0, n_z, zcopy, 0)
        plsc.subcore_barrier()

        crow = (c * 16 + s) * n_ch
        pltpu.sync_copy(sA_h.at[pl.ds(crow, n_ch)], iva)
        pltpu.sync_copy(dst_h.at[pl.ds(crow, n_ch)], ivd)

        def body(g, _):
            i0 = 2 * g
            d0 = pltpu.async_copy(tA_h.at[iva.at[i0]], ra0, sem)
            d1 = pltpu.async_copy(tA_h.at[iva.at[i0 + 1]], ra1, sem)
            d0.wait()
            pltpu.sync_copy(ra0, acc.at[ivd.at[i0]], add=True)
            d1.wait()
            pltpu.sync_copy(ra1, acc.at[ivd.at[i0 + 1]], add=True)
            return 0
        lax.fori_loop(0, n_ch // 2, body, 0)
        plsc.subcore_barrier()
        pltpu.sync_copy(acc.at[pl.ds(s * rows_per, rows_per)],
                        out_h.at[c, pl.ds(s * rows_per, rows_per)])

    return k(tA, sA.reshape(-1, CH), dst.reshape(-1, CH))


def _sc_segsum_range(tA, sA, tB, sB, dst, N):
    E = dst.shape[0]
    C = tA.shape[1]
    half = N // 2
    e_per_w = E // 16
    CHR = 64
    n_ch = e_per_w // CHR
    rows_per = half // 16
    n_z = rows_per // ZR

    scratch = [pltpu.VMEM((n_ch, CHR), jnp.int32),
               pltpu.VMEM((n_ch, CHR), jnp.int32),
               pltpu.VMEM((n_ch, CHR), jnp.int32),
               pltpu.VMEM((CHR, C), f32),
               pltpu.VMEM((CHR, C), f32),
               pltpu.VMEM((CHR, C), f32),
               pltpu.VMEM((CHR, C), f32),
               pltpu.VMEM((ZR, C), f32),
               pltpu.VMEM_SHARED((half + 8, C), f32),
               pltpu.SemaphoreType.DMA]

    @functools.partial(
        pl.kernel, mesh=_sc_mesh(),
        out_type=jax.ShapeDtypeStruct((N, C), f32),
        scratch_types=scratch)
    def k(tA_h, sA_h, tB_h, sB_h, dst_h, out_h,
          iva, ivb, ivd, ra0, ra1, rb0, rb1, zbuf, acc, sem):
        c = lax.axis_index("c")
        s = lax.axis_index("s")
        _zero_fill(zbuf, C)

        def zcopy(j, _):
            pltpu.sync_copy(zbuf, acc.at[pl.ds(s * rows_per + j * ZR, ZR)])
            return 0
        lax.fori_loop(0, n_z, zcopy, 0)
        plsc.subcore_barrier()

        lo = c * half
        crow = s * n_ch
        pltpu.sync_copy(sA_h.at[pl.ds(crow, n_ch)], iva)
        pltpu.sync_copy(sB_h.at[pl.ds(crow, n_ch)], ivb)
        pltpu.sync_copy(dst_h.at[pl.ds(crow, n_ch)], ivd)

        def remap(r, _):
            for kk in range(CHR // 16):
                sl = pl.ds(kk * 16, 16)
                v = ivd[r, sl] - lo
                inr = (v >= 0) & (v < half)
                ivd[r, sl] = jnp.where(inr, v, half)
            return 0
        lax.fori_loop(0, n_ch, remap, 0)

        def body(g, _):
            i0 = 2 * g
            d0a = pltpu.async_copy(tA_h.at[iva.at[i0]], ra0, sem)
            d0b = pltpu.async_copy(tB_h.at[ivb.at[i0]], rb0, sem)
            d1a = pltpu.async_copy(tA_h.at[iva.at[i0 + 1]], ra1, sem)
            d1b = pltpu.async_copy(tB_h.at[ivb.at[i0 + 1]], rb1, sem)
            d0a.wait()
            d0b.wait()
            pltpu.sync_copy(ra0, acc.at[ivd.at[i0]], add=True)
            pltpu.sync_copy(rb0, acc.at[ivd.at[i0]], add=True)
            d1a.wait()
            d1b.wait()
            pltpu.sync_copy(ra1, acc.at[ivd.at[i0 + 1]], add=True)
            pltpu.sync_copy(rb1, acc.at[ivd.at[i0 + 1]], add=True)
            return 0
        lax.fori_loop(0, n_ch // 2, body, 0)
        plsc.subcore_barrier()
        pltpu.sync_copy(acc.at[pl.ds(s * rows_per, rows_per)],
                        out_h.at[pl.ds(c * half + s * rows_per, rows_per)])

    return k(tA, sA.reshape(-1, CHR), tB, sB.reshape(-1, CHR), dst.reshape(-1, CHR))


def _sc_degree_part(dst, N):
    E = dst.shape[0]
    C = 128
    e_per_w = E // NW
    n_ch = e_per_w // CH
    rows_per = N // 16
    n_z = rows_per // ZR

    scratch = [pltpu.VMEM((CH,), jnp.int32),
               pltpu.VMEM((CH, C), f32),
               pltpu.VMEM((ZR, C), f32),
               pltpu.VMEM_SHARED((N, C), f32),
               pltpu.SemaphoreType.DMA]

    @functools.partial(
        pl.kernel, mesh=_sc_mesh(),
        out_type=jax.ShapeDtypeStruct((2, N, C), f32),
        scratch_types=scratch)
    def k(dst_h, out_h, ivd, obuf, zbuf, acc, sem):
        c = lax.axis_index("c")
        s = lax.axis_index("s")
        _zero_fill(zbuf, C)

        def fill(r, _):
            for kk in range(C // 16):
                obuf[r, pl.ds(kk * 16, 16)] = jnp.ones((16,), f32)
            return 0
        lax.fori_loop(0, ZR, fill, 0)

        def zcopy(j, _):
            pltpu.sync_copy(zbuf, acc.at[pl.ds(s * rows_per + j * ZR, ZR)])
            return 0
        lax.fori_loop(0, n_z, zcopy, 0)
        plsc.subcore_barrier()

        base = (c * 16 + s) * e_per_w

        def body(i, _):
            off = base + i * CH
            pltpu.sync_copy(dst_h.at[pl.ds(off, CH)], ivd)
            pltpu.sync_copy(obuf, acc.at[ivd], add=True)
            return 0
        lax.fori_loop(0, n_ch, body, 0)
        plsc.subcore_barrier()
        pltpu.sync_copy(acc.at[pl.ds(s * rows_per, rows_per)],
                        out_h.at[c, pl.ds(s * rows_per, rows_per)])

    return k(dst)


def _sc_degree_range(dst, N):
    E = dst.shape[0]
    C = 128
    half = N // 2
    e_per_w = E // 16
    n_ch = e_per_w // CH
    rows_per = half // 16
    n_z = rows_per // ZR

    scratch = [pltpu.VMEM((CH,), jnp.int32),
               pltpu.VMEM((CH, C), f32),
               pltpu.VMEM((ZR, C), f32),
               pltpu.VMEM_SHARED((half + 8, C), f32),
               pltpu.SemaphoreType.DMA]

    @functools.partial(
        pl.kernel, mesh=_sc_mesh(),
        out_type=jax.ShapeDtypeStruct((N, C), f32),
        scratch_types=scratch)
    def k(dst_h, out_h, ivd, obuf, zbuf, acc, sem):
        c = lax.axis_index("c")
        s = lax.axis_index("s")
        _zero_fill(zbuf, C)

        def fill(r, _):
            for kk in range(C // 16):
                obuf[r, pl.ds(kk * 16, 16)] = jnp.ones((16,), f32)
            return 0
        lax.fori_loop(0, ZR, fill, 0)

        def zcopy(j, _):
            pltpu.sync_copy(zbuf, acc.at[pl.ds(s * rows_per + j * ZR, ZR)])
            return 0
        lax.fori_loop(0, n_z, zcopy, 0)
        plsc.subcore_barrier()

        lo = c * half
        base = s * e_per_w

        def body(i, _):
            off = base + i * CH
            pltpu.sync_copy(dst_h.at[pl.ds(off, CH)], ivd)
            for kk in range(CH // 16):
                sl = pl.ds(kk * 16, 16)
                v = ivd[sl] - lo
                inr = (v >= 0) & (v < half)
                ivd[sl] = jnp.where(inr, v, half)
            pltpu.sync_copy(obuf, acc.at[ivd], add=True)
            return 0
        lax.fori_loop(0, n_ch, body, 0)
        plsc.subcore_barrier()
        pltpu.sync_copy(acc.at[pl.ds(s * rows_per, rows_per)],
                        out_h.at[pl.ds(c * half + s * rows_per, rows_per)])

    return k(dst)


def kernel(ent_embeds, rel_embeds, word_embeds, W_gcn1, W_gcn2, W_comp1, W_loop1,
           W_rel1, W_comp2, W_loop2, W_rel2, W_attn, W_out,
           node_ids, edge_src, edge_dst, edge_type, node_graph_ids, edge_graph_ids,
           wg_node_ids, wg_src, wg_dst, wg_graph_ids, word_query_idx, time_idx):
    wp = jnp.pad(word_embeds, ((0, 0), (0, 28)))
    Wg1p = jnp.pad(W_gcn1, ((0, 28), (0, 0)))
    ent_c = [ent_embeds[:, 128 * k:128 * (k + 1)] for k in range(2)]
    node_gidf = node_graph_ids.astype(f32)[:, None]
    edge_gidf = edge_graph_ids.astype(f32)[:, None]
    wg_gidf = wg_graph_ids.astype(f32)[:, None]
    onehot = (time_idx.reshape(-1, 1) == jnp.arange(T)[None, :]).astype(f32)
    wqi = word_query_idx.reshape(-1)

    hc = _sc_gather_multi(ent_c, node_ids)
    wh0 = _sc_gather(wp, wg_node_ids)
    wdegp = _sc_degree_part(wg_dst, WN)
    ndeg = _sc_degree_range(edge_dst, NN)

    wa0 = _sc_segsum_part(wh0, wg_src, wg_dst, WN)
    wh1c0, wh1c1 = _word_layer1(wa0, wh0, wdegp, Wg1p)
    wa1_0 = _sc_segsum_part(wh1c0, wg_src, wg_dst, WN)
    wa1_1 = _sc_segsum_part(wh1c1, wg_src, wg_dst, WN)
    wh2c0, wh2c1, gw = _word_layer2(wa1_0, wa1_1, wh1c0, wh1c1, wdegp, W_gcn2, wg_gidf)

    nr0, nr1, ne1, E2t = _rel_tables(rel_embeds, W_rel1, W_rel2)

    s1c0 = _sc_segsum_range(hc[0], edge_src, nr0, edge_type, edge_dst, NN)
    s1c1 = _sc_segsum_range(hc[1], edge_src, nr1, edge_type, edge_dst, NN)
    h1 = _node_layer1(s1c0, s1c1, ndeg, hc[0], hc[1], W_comp1, W_loop1)
    s2 = _sc_segsum_range(h1, edge_src, ne1, edge_type, edge_dst, NN)
    h2 = _node_layer2(s2, ndeg, h1, W_comp2, W_loop2)
    e2 = _sc_gather(E2t, edge_type)

    HloN, HhiN = _sc_gather_multi([wh2c0, wh2c1], wqi[:4 * NN])
    HloE, HhiE = _sc_gather_multi([wh2c0, wh2c1], wqi[4 * NN:])
    gn = _attn_gmax(h2, HloN, HhiN, node_gidf, W_attn, W_out, 0)
    ge = _attn_gmax(e2, HloE, HhiE, edge_gidf, W_attn, W_out, 0)

    out640 = _final(gn, ge, gw, onehot)
    return out640.reshape(time_idx.shape[0], time_idx.shape[1], 3 * 256)


# speedup vs baseline: 1.1096x; 1.0314x over previous
import functools
import jax
import jax.numpy as jnp
from jax import lax
from jax.experimental import pallas as pl
from jax.experimental.pallas import tpu as pltpu
from jax.experimental.pallas import tpu_sc as plsc

f32 = jnp.float32
NN = 16384
NE = 32768
WN = 8192
WE = 32768
T = 16
R = 512

_INTERPRET = False


def _tc(body, out_shape, grid, in_specs, out_specs, scratch_shapes=()):
    return pl.pallas_call(
        body, out_shape=out_shape, grid=grid,
        in_specs=in_specs, out_specs=out_specs,
        scratch_shapes=list(scratch_shapes),
        interpret=_INTERPRET)


def _rowmap(i):
    return (i, 0)


def _partmap(i):
    return (0, i, 0)


def _constmap(*_):
    return (0, 0)


def _word_layer1(aggp, wh0, wdegp, Wg1p):
    def body(a_ref, w_ref, d_ref, wg_ref, o0_ref, o1_ref):
        deg = d_ref[0, :, 0:1] + d_ref[1, :, 0:1] + 1.0
        x = (a_ref[0] + a_ref[1] + w_ref[...]) / deg
        y = jnp.maximum(jnp.dot(x, wg_ref[...], preferred_element_type=f32), 0.0)
        o0_ref[...] = y[:, :128]
        o1_ref[...] = y[:, 128:]
    nb = WN // R
    return _tc(
        body,
        (jax.ShapeDtypeStruct((WN, 128), f32), jax.ShapeDtypeStruct((WN, 128), f32)),
        (nb,),
        [pl.BlockSpec((2, R, 128), _partmap),
         pl.BlockSpec((R, 128), _rowmap),
         pl.BlockSpec((2, R, 128), _partmap),
         pl.BlockSpec((128, 256), _constmap)],
        (pl.BlockSpec((R, 128), _rowmap), pl.BlockSpec((R, 128), _rowmap)),
    )(aggp, wh0, wdegp, Wg1p)


def _word_layer2(a0, a1, c0, c1, wdegp, Wg2, wgidf):
    def body(a0_ref, a1_ref, c0_ref, c1_ref, d_ref, wg_ref, g_ref,
             o0_ref, o1_ref, gw_ref, acc_ref):
        i = pl.program_id(0)
        deg = d_ref[0, :, 0:1] + d_ref[1, :, 0:1] + 1.0
        x0 = (a0_ref[0] + a0_ref[1] + c0_ref[...]) / deg
        x1 = (a1_ref[0] + a1_ref[1] + c1_ref[...]) / deg
        y = jnp.maximum(
            jnp.dot(x0, wg_ref[0:128, :], preferred_element_type=f32)
            + jnp.dot(x1, wg_ref[128:256, :], preferred_element_type=f32), 0.0)
        o0_ref[...] = y[:, :128]
        o1_ref[...] = y[:, 128:]

        @pl.when(i == 0)
        def _():
            acc_ref[...] = jnp.full((T, 256), -1e38, f32)
        g = g_ref[...]
        gmin = jnp.min(g)
        gmax = jnp.max(g)
        for t in range(T):
            @pl.when(jnp.logical_and(gmin <= float(t), float(t) <= gmax))
            def _():
                m = g == float(t)
                acc_ref[t:t + 1, :] = jnp.maximum(
                    acc_ref[t:t + 1, :],
                    jnp.max(jnp.where(m, y, -1e38), axis=0, keepdims=True))

        @pl.when(i == pl.num_programs(0) - 1)
        def _():
            gw_ref[...] = jnp.where(acc_ref[...] < -1.0, 0.0, acc_ref[...])
    nb = WN // R
    return _tc(
        body,
        (jax.ShapeDtypeStruct((WN, 128), f32), jax.ShapeDtypeStruct((WN, 128), f32),
         jax.ShapeDtypeStruct((T, 256), f32)),
        (nb,),
        [pl.BlockSpec((2, R, 128), _partmap),
         pl.BlockSpec((2, R, 128), _partmap),
         pl.BlockSpec((R, 128), _rowmap),
         pl.BlockSpec((R, 128), _rowmap),
         pl.BlockSpec((2, R, 128), _partmap),
         pl.BlockSpec((256, 256), _constmap),
         pl.BlockSpec((R, 1), _rowmap)],
        (pl.BlockSpec((R, 128), _rowmap), pl.BlockSpec((R, 128), _rowmap),
         pl.BlockSpec((T, 256), _constmap)),
        [pltpu.VMEM((T, 256), f32)],
    )(a0, a1, c0, c1, wdegp, Wg2, wgidf)


def _rel_tables(rel_embeds, Wr1, Wr2):
    def body(rel_ref, w1_ref, w2_ref, nr0, nr1, ne_ref, e2_ref):
        rel = rel_ref[...]
        E1 = jnp.maximum(jnp.dot(rel, w1_ref[...], preferred_element_type=f32), 0.0)
        E2 = jnp.maximum(jnp.dot(E1, w2_ref[...], preferred_element_type=f32), 0.0)
        nr0[...] = -rel[:, 0:128]
        nr1[...] = -rel[:, 128:256]
        ne_ref[...] = -E1
        e2_ref[...] = E2
    shp = jax.ShapeDtypeStruct((256, 128), f32)
    return _tc(
        body,
        (shp, shp, shp, jax.ShapeDtypeStruct((256, 256), f32)),
        (1,),
        [pl.BlockSpec((256, 256), _constmap),
         pl.BlockSpec((256, 128), _constmap),
         pl.BlockSpec((128, 256), _constmap)],
        (pl.BlockSpec((256, 128), _constmap),) * 3
        + (pl.BlockSpec((256, 256), _constmap),),
    )(rel_embeds, Wr1, Wr2)


def _node_layer1(s1c0, s1c1, ndeg, hc0, hc1, Wc1, Wl1):
    def body(p0, p1, d_ref, h0, h1, wc_ref, wl_ref, o_ref):
        deg = d_ref[:, 0:1] + 1.0
        S = jnp.concatenate([p0[...], p1[...]], axis=1)
        hh = jnp.concatenate([h0[...], h1[...]], axis=1)
        agg = jnp.dot(S, wc_ref[...], preferred_element_type=f32) / deg
        o_ref[...] = jnp.maximum(
            agg + jnp.dot(hh, wl_ref[...], preferred_element_type=f32), 0.0)
    nb = NN // R
    return _tc(
        body,
        jax.ShapeDtypeStruct((NN, 128), f32),
        (nb,),
        [pl.BlockSpec((R, 128), _rowmap)] * 2
        + [pl.BlockSpec((R, 128), _rowmap)]
        + [pl.BlockSpec((R, 128), _rowmap)] * 2
        + [pl.BlockSpec((256, 128), _constmap), pl.BlockSpec((256, 128), _constmap)],
        pl.BlockSpec((R, 128), _rowmap),
    )(s1c0, s1c1, ndeg, hc0, hc1, Wc1, Wl1)


def _node_layer2(s2, ndeg, h1, Wc2, Wl2):
    def body(q_ref, d_ref, h_ref, wc_ref, wl_ref, o_ref):
        deg = d_ref[:, 0:1] + 1.0
        agg = jnp.dot(q_ref[...], wc_ref[...], preferred_element_type=f32) / deg
        o_ref[...] = jnp.maximum(
            agg + jnp.dot(h_ref[...], wl_ref[...], preferred_element_type=f32), 0.0)
    nb = NN // R
    return _tc(
        body,
        jax.ShapeDtypeStruct((NN, 256), f32),
        (nb,),
        [pl.BlockSpec((R, 128), _rowmap)] * 3
        + [pl.BlockSpec((128, 256), _constmap), pl.BlockSpec((128, 256), _constmap)],
        pl.BlockSpec((R, 256), _rowmap),
    )(s2, ndeg, h1, Wc2, Wl2)


def _attn_gmax(Q, Hlo, Hhi, gidf, Wa, Wo, q_off, nb):
    def qmap(i):
        return (i + q_off, 0)

    def hmmap(i):
        return (i, 0)

    def body(q_ref, hlo_ref, hhi_ref, g_ref, wa_ref, wo_ref, out_ref, acc_ref):
        i = pl.program_id(0)
        Qb = q_ref[...]
        QW = jnp.dot(Qb, wa_ref[...], preferred_element_type=f32)
        Hlo_b = hlo_ref[...].reshape(R, 4, 128)
        Hhi_b = hhi_ref[...].reshape(R, 4, 128)
        QWlo = QW[:, :128]
        QWhi = QW[:, 128:]
        sc = [jnp.sum(QWlo * Hlo_b[:, m, :], axis=1, keepdims=True)
              + jnp.sum(QWhi * Hhi_b[:, m, :], axis=1, keepdims=True)
              for m in range(4)]
        smax = jnp.maximum(jnp.maximum(sc[0], sc[1]), jnp.maximum(sc[2], sc[3]))
        ew = [jnp.exp(s - smax) for s in sc]
        tot = ew[0] + ew[1] + ew[2] + ew[3]
        ctx_lo = sum((ew[m] / tot) * Hlo_b[:, m, :] for m in range(4))
        ctx_hi = sum((ew[m] / tot) * Hhi_b[:, m, :] for m in range(4))
        y = jnp.tanh(
            jnp.dot(ctx_lo, wo_ref[0:128, :], preferred_element_type=f32)
            + jnp.dot(ctx_hi, wo_ref[128:256, :], preferred_element_type=f32)
            + jnp.dot(Qb, wo_ref[256:512, :], preferred_element_type=f32))

        @pl.when(i == 0)
        def _():
            acc_ref[...] = jnp.full((T, 256), -1e38, f32)
        g = g_ref[...]
        gmin = jnp.min(g)
        gmax = jnp.max(g)
        for t in range(T):
            @pl.when(jnp.logical_and(gmin <= float(t), float(t) <= gmax))
            def _():
                m = g == float(t)
                acc_ref[t:t + 1, :] = jnp.maximum(
                    acc_ref[t:t + 1, :],
                    jnp.max(jnp.where(m, y, -1e38), axis=0, keepdims=True))

        @pl.when(i == pl.num_programs(0) - 1)
        def _():
            out_ref[...] = acc_ref[...]

    return _tc(
        body,
        jax.ShapeDtypeStruct((T, 256), f32),
        (nb,),
        [pl.BlockSpec((R, 256), qmap),
         pl.BlockSpec((4 * R, 128), hmmap),
         pl.BlockSpec((4 * R, 128), hmmap),
         pl.BlockSpec((R, 1), qmap),
         pl.BlockSpec((256, 256), _constmap),
         pl.BlockSpec((512, 256), _constmap)],
        pl.BlockSpec((T, 256), _constmap),
        [pltpu.VMEM((T, 256), f32)],
    )(Q, Hlo, Hhi, gidf, Wa, Wo)


def _final(gn, geA, geB, gw, onehot):
    def body(oh_ref, gn_ref, gea_ref, geb_ref, gw_ref, o_ref):
        oh = oh_ref[...]
        gn_v = gn_ref[...]
        gn_v = jnp.where(gn_v < -2.0, 0.0, gn_v)
        ge_v = jnp.maximum(gea_ref[...], geb_ref[...])
        ge_v = jnp.where(ge_v < -2.0, 0.0, ge_v)
        o_ref[:, 0:256] = jnp.dot(oh, gn_v, preferred_element_type=f32)
        o_ref[:, 256:512] = jnp.dot(oh, ge_v, preferred_element_type=f32)
        o_ref[:, 512:768] = jnp.dot(oh, gw_ref[...], preferred_element_type=f32)
    return _tc(
        body,
        jax.ShapeDtypeStruct((640, 768), f32),
        (1,),
        [pl.BlockSpec((640, 16), _constmap)]
        + [pl.BlockSpec((T, 256), _constmap)] * 4,
        pl.BlockSpec((640, 768), _constmap),
    )(onehot, gn, geA, geB, gw)


NW = 32
CH = 128
ZR = 128


def _sc_mesh():
    return plsc.VectorSubcoreMesh(core_axis_name="c", subcore_axis_name="s")


def _zero_fill(zbuf, C):
    def zrow(r, _):
        for kk in range(C // 16):
            zbuf[r, pl.ds(kk * 16, 16)] = jnp.zeros((16,), f32)
        return 0
    lax.fori_loop(0, ZR, zrow, 0)


def _sc_gather_multi(tables, idx):
    B = idx.shape[0]
    b_per_w = B // NW
    n_ch = b_per_w // CH
    nt = len(tables)
    assert n_ch % 2 == 0
    idx2d = idx.reshape(B // CH, CH)
    scratch = [pltpu.VMEM((n_ch, CH), jnp.int32)]
    scratch += [pltpu.VMEM((2, CH, t.shape[1]), f32) for t in tables]
    scratch += [pltpu.SemaphoreType.DMA] * 4

    @functools.partial(
        pl.kernel, mesh=_sc_mesh(),
        out_type=tuple(jax.ShapeDtypeStruct((B, t.shape[1]), f32) for t in tables),
        scratch_types=scratch)
    def k(*refs):
        tbls = refs[:nt]
        idx_hbm = refs[nt]
        outs = refs[nt + 1:2 * nt + 1]
        idx_v = refs[2 * nt + 1]
        rows = refs[2 * nt + 2:3 * nt + 2]
        gsem = refs[3 * nt + 2:3 * nt + 4]
        wsem = refs[3 * nt + 4:3 * nt + 6]
        wid = lax.axis_index("s") * 2 + lax.axis_index("c")
        pltpu.sync_copy(idx_hbm.at[pl.ds(wid * n_ch, n_ch)], idx_v)

        def fire_g(i, b):
            for j in range(nt):
                pltpu.async_copy(tbls[j].at[idx_v.at[i]], rows[j].at[b], gsem[b])

        def drain_g(b):
            for j in range(nt):
                pltpu.make_async_copy(tbls[j].at[idx_v.at[0]],
                                      rows[j].at[b], gsem[b]).wait()

        def fire_w(i, b):
            off = (wid * n_ch + i) * CH
            for j in range(nt):
                pltpu.async_copy(rows[j].at[b], outs[j].at[pl.ds(off, CH)], wsem[b])

        def drain_w(b):
            for j in range(nt):
                pltpu.make_async_copy(rows[j].at[b],
                                      outs[j].at[pl.ds(0, CH)], wsem[b]).wait()

        fire_g(0, 0)

        def body(hh, _):
            g0 = 2 * hh
            fire_g(g0 + 1, 1)
            drain_g(0)
            fire_w(g0, 0)
            drain_g(1)
            fire_w(g0 + 1, 1)
            drain_w(0)

            @pl.when(hh + 1 < n_ch // 2)
            def _():
                fire_g(g0 + 2, 0)
            drain_w(1)
            return 0
        lax.fori_loop(0, n_ch // 2, body, 0)

    res = k(*tables, idx2d)
    return list(res) if isinstance(res, (tuple, list)) else [res]


def _sc_gather(table, idx):
    return _sc_gather_multi([table], idx)[0]


def _sc_segsum_part(tA, sA, dst, N):
    E = dst.shape[0]
    C = tA.shape[1]
    e_per_w = E // NW
    n_ch = e_per_w // CH
    rows_per = N // 16
    n_z = rows_per // ZR

    scratch = [pltpu.VMEM((n_ch, CH), jnp.int32),
               pltpu.VMEM((n_ch, CH), jnp.int32),
               pltpu.VMEM((CH, C), f32),
               pltpu.VMEM((CH, C), f32),
               pltpu.VMEM((ZR, C), f32),
               pltpu.VMEM_SHARED((N, C), f32),
               pltpu.SemaphoreType.DMA]

    @functools.partial(
        pl.kernel, mesh=_sc_mesh(),
        out_type=jax.ShapeDtypeStruct((2, N, C), f32),
        scratch_types=scratch)
    def k(tA_h, sA_h, dst_h, out_h, iva, ivd, ra0, ra1, zbuf, acc, sem):
        c = lax.axis_index("c")
        s = lax.axis_index("s")
        _zero_fill(zbuf, C)

        def zcopy(j, _):
            pltpu.sync_copy(zbuf, acc.at[pl.ds(s * rows_per + j * ZR, ZR)])
            return 0
        lax.fori_loop(0, n_z, zcopy, 0)
        plsc.subcore_barrier()

        crow = (c * 16 + s) * n_ch
        pltpu.sync_copy(sA_h.at[pl.ds(crow, n_ch)], iva)
        pltpu.sync_copy(dst_h.at[pl.ds(crow, n_ch)], ivd)

        def body(g, _):
            i0 = 2 * g
            d0 = pltpu.async_copy(tA_h.at[iva.at[i0]], ra0, sem)
            d1 = pltpu.async_copy(tA_h.at[iva.at[i0 + 1]], ra1, sem)
            d0.wait()
            pltpu.sync_copy(ra0, acc.at[ivd.at[i0]], add=True)
            d1.wait()
            pltpu.sync_copy(ra1, acc.at[ivd.at[i0 + 1]], add=True)
            return 0
        lax.fori_loop(0, n_ch // 2, body, 0)
        plsc.subcore_barrier()
        pltpu.sync_copy(acc.at[pl.ds(s * rows_per, rows_per)],
                        out_h.at[c, pl.ds(s * rows_per, rows_per)])

    return k(tA, sA.reshape(-1, CH), dst.reshape(-1, CH))


def _sc_segsum_range(tA, sA, tB, sB, dst, N):
    E = dst.shape[0]
    C = tA.shape[1]
    half = N // 2
    e_per_w = E // 16
    CHR = 64
    n_ch = e_per_w // CHR
    rows_per = half // 16
    n_z = rows_per // ZR

    scratch = [pltpu.VMEM((n_ch, CHR), jnp.int32),
               pltpu.VMEM((n_ch, CHR), jnp.int32),
               pltpu.VMEM((n_ch, CHR), jnp.int32),
               pltpu.VMEM((CHR, C), f32),
               pltpu.VMEM((CHR, C), f32),
               pltpu.VMEM((CHR, C), f32),
               pltpu.VMEM((CHR, C), f32),
               pltpu.VMEM((ZR, C), f32),
               pltpu.VMEM_SHARED((half + 8, C), f32),
               pltpu.SemaphoreType.DMA]

    @functools.partial(
        pl.kernel, mesh=_sc_mesh(),
        out_type=jax.ShapeDtypeStruct((N, C), f32),
        scratch_types=scratch)
    def k(tA_h, sA_h, tB_h, sB_h, dst_h, out_h,
          iva, ivb, ivd, ra0, ra1, rb0, rb1, zbuf, acc, sem):
        c = lax.axis_index("c")
        s = lax.axis_index("s")
        _zero_fill(zbuf, C)

        def zcopy(j, _):
            pltpu.sync_copy(zbuf, acc.at[pl.ds(s * rows_per + j * ZR, ZR)])
            return 0
        lax.fori_loop(0, n_z, zcopy, 0)
        plsc.subcore_barrier()

        lo = c * half
        crow = s * n_ch
        pltpu.sync_copy(sA_h.at[pl.ds(crow, n_ch)], iva)
        pltpu.sync_copy(sB_h.at[pl.ds(crow, n_ch)], ivb)
        pltpu.sync_copy(dst_h.at[pl.ds(crow, n_ch)], ivd)

        def remap(r, _):
            for kk in range(CHR // 16):
                sl = pl.ds(kk * 16, 16)
                v = ivd[r, sl] - lo
                inr = (v >= 0) & (v < half)
                ivd[r, sl] = jnp.where(inr, v, half)
            return 0
        lax.fori_loop(0, n_ch, remap, 0)

        def body(g, _):
            i0 = 2 * g
            d0a = pltpu.async_copy(tA_h.at[iva.at[i0]], ra0, sem)
            d0b = pltpu.async_copy(tB_h.at[ivb.at[i0]], rb0, sem)
            d1a = pltpu.async_copy(tA_h.at[iva.at[i0 + 1]], ra1, sem)
            d1b = pltpu.async_copy(tB_h.at[ivb.at[i0 + 1]], rb1, sem)
            d0a.wait()
            d0b.wait()
            pltpu.sync_copy(ra0, acc.at[ivd.at[i0]], add=True)
            pltpu.sync_copy(rb0, acc.at[ivd.at[i0]], add=True)
            d1a.wait()
            d1b.wait()
            pltpu.sync_copy(ra1, acc.at[ivd.at[i0 + 1]], add=True)
            pltpu.sync_copy(rb1, acc.at[ivd.at[i0 + 1]], add=True)
            return 0
        lax.fori_loop(0, n_ch // 2, body, 0)
        plsc.subcore_barrier()
        pltpu.sync_copy(acc.at[pl.ds(s * rows_per, rows_per)],
                        out_h.at[pl.ds(c * half + s * rows_per, rows_per)])

    return k(tA, sA.reshape(-1, CHR), tB, sB.reshape(-1, CHR), dst.reshape(-1, CHR))


def _sc_degree_part(dst, N):
    E = dst.shape[0]
    C = 128
    e_per_w = E // NW
    n_ch = e_per_w // CH
    rows_per = N // 16
    n_z = rows_per // ZR

    scratch = [pltpu.VMEM((CH,), jnp.int32),
               pltpu.VMEM((CH, C), f32),
               pltpu.VMEM((ZR, C), f32),
               pltpu.VMEM_SHARED((N, C), f32),
               pltpu.SemaphoreType.DMA]

    @functools.partial(
        pl.kernel, mesh=_sc_mesh(),
        out_type=jax.ShapeDtypeStruct((2, N, C), f32),
        scratch_types=scratch)
    def k(dst_h, out_h, ivd, obuf, zbuf, acc, sem):
        c = lax.axis_index("c")
        s = lax.axis_index("s")
        _zero_fill(zbuf, C)

        def fill(r, _):
            for kk in range(C // 16):
                obuf[r, pl.ds(kk * 16, 16)] = jnp.ones((16,), f32)
            return 0
        lax.fori_loop(0, ZR, fill, 0)

        def zcopy(j, _):
            pltpu.sync_copy(zbuf, acc.at[pl.ds(s * rows_per + j * ZR, ZR)])
            return 0
        lax.fori_loop(0, n_z, zcopy, 0)
        plsc.subcore_barrier()

        base = (c * 16 + s) * e_per_w

        def body(i, _):
            off = base + i * CH
            pltpu.sync_copy(dst_h.at[pl.ds(off, CH)], ivd)
            pltpu.sync_copy(obuf, acc.at[ivd], add=True)
            return 0
        lax.fori_loop(0, n_ch, body, 0)
        plsc.subcore_barrier()
        pltpu.sync_copy(acc.at[pl.ds(s * rows_per, rows_per)],
                        out_h.at[c, pl.ds(s * rows_per, rows_per)])

    return k(dst)


def _sc_degree_range(dst, N):
    E = dst.shape[0]
    C = 128
    half = N // 2
    e_per_w = E // 16
    n_ch = e_per_w // CH
    rows_per = half // 16
    n_z = rows_per // ZR

    scratch = [pltpu.VMEM((CH,), jnp.int32),
               pltpu.VMEM((CH, C), f32),
               pltpu.VMEM((ZR, C), f32),
               pltpu.VMEM_SHARED((half + 8, C), f32),
               pltpu.SemaphoreType.DMA]

    @functools.partial(
        pl.kernel, mesh=_sc_mesh(),
        out_type=jax.ShapeDtypeStruct((N, C), f32),
        scratch_types=scratch)
    def k(dst_h, out_h, ivd, obuf, zbuf, acc, sem):
        c = lax.axis_index("c")
        s = lax.axis_index("s")
        _zero_fill(zbuf, C)

        def fill(r, _):
            for kk in range(C // 16):
                obuf[r, pl.ds(kk * 16, 16)] = jnp.ones((16,), f32)
            return 0
        lax.fori_loop(0, ZR, fill, 0)

        def zcopy(j, _):
            pltpu.sync_copy(zbuf, acc.at[pl.ds(s * rows_per + j * ZR, ZR)])
            return 0
        lax.fori_loop(0, n_z, zcopy, 0)
        plsc.subcore_barrier()

        lo = c * half
        base = s * e_per_w

        def body(i, _):
            off = base + i * CH
            pltpu.sync_copy(dst_h.at[pl.ds(off, CH)], ivd)
            for kk in range(CH // 16):
                sl = pl.ds(kk * 16, 16)
                v = ivd[sl] - lo
                inr = (v >= 0) & (v < half)
                ivd[sl] = jnp.where(inr, v, half)
            pltpu.sync_copy(obuf, acc.at[ivd], add=True)
            return 0
        lax.fori_loop(0, n_ch, body, 0)
        plsc.subcore_barrier()
        pltpu.sync_copy(acc.at[pl.ds(s * rows_per, rows_per)],
                        out_h.at[pl.ds(c * half + s * rows_per, rows_per)])

    return k(dst)


def kernel(ent_embeds, rel_embeds, word_embeds, W_gcn1, W_gcn2, W_comp1, W_loop1,
           W_rel1, W_comp2, W_loop2, W_rel2, W_attn, W_out,
           node_ids, edge_src, edge_dst, edge_type, node_graph_ids, edge_graph_ids,
           wg_node_ids, wg_src, wg_dst, wg_graph_ids, word_query_idx, time_idx):
    wp = jnp.pad(word_embeds, ((0, 0), (0, 28)))
    Wg1p = jnp.pad(W_gcn1, ((0, 28), (0, 0)))
    ent_c = [ent_embeds[:, 128 * k:128 * (k + 1)] for k in range(2)]
    node_gidf = node_graph_ids.astype(f32)[:, None]
    edge_gidf = edge_graph_ids.astype(f32)[:, None]
    wg_gidf = wg_graph_ids.astype(f32)[:, None]
    onehot = (time_idx.reshape(-1, 1) == jnp.arange(T)[None, :]).astype(f32)
    wqi = word_query_idx.reshape(-1)

    hc = _sc_gather_multi(ent_c, node_ids)
    wh0 = _sc_gather(wp, wg_node_ids)
    wdegp = _sc_degree_part(wg_dst, WN)
    ndeg = _sc_degree_range(edge_dst, NN)

    wa0 = _sc_segsum_part(wh0, wg_src, wg_dst, WN)
    wh1c0, wh1c1 = _word_layer1(wa0, wh0, wdegp, Wg1p)
    wa1_0 = _sc_segsum_part(wh1c0, wg_src, wg_dst, WN)
    wa1_1 = _sc_segsum_part(wh1c1, wg_src, wg_dst, WN)
    wh2c0, wh2c1, gw = _word_layer2(wa1_0, wa1_1, wh1c0, wh1c1, wdegp, W_gcn2, wg_gidf)

    nr0, nr1, ne1, E2t = _rel_tables(rel_embeds, W_rel1, W_rel2)

    s1c0 = _sc_segsum_range(hc[0], edge_src, nr0, edge_type, edge_dst, NN)
    s1c1 = _sc_segsum_range(hc[1], edge_src, nr1, edge_type, edge_dst, NN)
    h1 = _node_layer1(s1c0, s1c1, ndeg, hc[0], hc[1], W_comp1, W_loop1)
    s2 = _sc_segsum_range(h1, edge_src, ne1, edge_type, edge_dst, NN)
    h2 = _node_layer2(s2, ndeg, h1, W_comp2, W_loop2)
    e2 = _sc_gather(E2t, edge_type)

    wqiE = wqi[4 * NN:]
    HloN, HhiN = _sc_gather_multi([wh2c0, wh2c1], wqi[:4 * NN])
    HloEa, HhiEa = _sc_gather_multi([wh2c0, wh2c1], wqiE[:4 * (NE // 2)])
    HloEb, HhiEb = _sc_gather_multi([wh2c0, wh2c1], wqiE[4 * (NE // 2):])
    nbE = (NE // 2) // R
    gn = _attn_gmax(h2, HloN, HhiN, node_gidf, W_attn, W_out, 0, NN // R)
    geA = _attn_gmax(e2, HloEa, HhiEa, edge_gidf, W_attn, W_out, 0, nbE)
    geB = _attn_gmax(e2, HloEb, HhiEb, edge_gidf, W_attn, W_out, nbE, nbE)

    out640 = _final(gn, geA, geB, gw, onehot)
    return out640.reshape(time_idx.shape[0], time_idx.shape[1], 3 * 256)
